# baseline (speedup 1.0000x reference)
import functools

import jax
import jax.numpy as jnp
from jax.experimental import pallas as pl
from jax.experimental.pallas import tpu as pltpu

_TB = 1024


def _cdiv(a: int, b: int) -> int:
    return (a + b - 1) // b


def _mlp_kernel(x_ref, w1_ref, b1_ref, w2_ref, b2_ref, o_ref,
                w1b_ref, w2b_ref):
    i = pl.program_id(0)

    def body(w1b, w2b):
        xb = x_ref[...].astype(jnp.bfloat16)
        h = jnp.dot(xb, w1b, preferred_element_type=jnp.float32)
        hb = jnp.maximum(h + b1_ref[...], 0.0).astype(jnp.bfloat16)
        y = jnp.dot(hb, w2b, preferred_element_type=jnp.float32)
        o_ref[...] = (y + b2_ref[...]).astype(o_ref.dtype)

    @pl.when(i == 0)
    def _():
        w1b = w1_ref[...].astype(jnp.bfloat16)
        w2b = w2_ref[...].astype(jnp.bfloat16)
        w1b_ref[...] = w1b
        w2b_ref[...] = w2b
        body(w1b, w2b)

    @pl.when(i > 0)
    def _():
        body(w1b_ref[...], w2b_ref[...])


@jax.jit
def kernel(x, w1_t, b1_r, w2_t, b2_r):
    b, n_feature = x.shape
    n_hidden, n_output = w2_t.shape
    b1_f = b1_r.astype(jnp.float32)
    b2_f = b2_r.astype(jnp.float32)

    tb = min(_TB, max(8, _cdiv(b, 8) * 8))
    nb = _cdiv(b, tb)
    b_pad = nb * tb
    if b_pad != b:
        x = jnp.pad(x, ((0, b_pad - b), (0, 0)))

    out = pl.pallas_call(
        _mlp_kernel,
        out_shape=jax.ShapeDtypeStruct((b_pad, n_output), x.dtype),
        grid=(nb,),
        in_specs=[
            pl.BlockSpec((tb, n_feature), lambda i: (i, 0)),
            pl.BlockSpec((n_feature, n_hidden), lambda i: (0, 0)),
            pl.BlockSpec((1, n_hidden), lambda i: (0, 0)),
            pl.BlockSpec((n_hidden, n_output), lambda i: (0, 0)),
            pl.BlockSpec((1, n_output), lambda i: (0, 0)),
        ],
        out_specs=pl.BlockSpec((tb, n_output), lambda i: (i, 0)),
        scratch_shapes=[
            pltpu.VMEM((n_feature, n_hidden), jnp.bfloat16),
            pltpu.VMEM((n_hidden, n_output), jnp.bfloat16),
        ],
        compiler_params=pltpu.CompilerParams(
            dimension_semantics=("arbitrary",),
            vmem_limit_bytes=int(64 * 1024 * 1024 * 0.92)),
    )(x, w1_t, b1_f, w2_t, b2_f)

    if b_pad != b:
        out = out[:b]
    return out


# speedup vs baseline: 1.0068x; 1.0068x over previous
import functools

import jax
import jax.numpy as jnp
from jax.experimental import pallas as pl
from jax.experimental.pallas import tpu as pltpu

_TB = 1024


def _cdiv(a: int, b: int) -> int:
    return (a + b - 1) // b


def _mlp_kernel(x_ref, w1_ref, b1_ref, w2_ref, b2_ref, o_ref):
    xb = x_ref[...].astype(jnp.bfloat16)
    w1b = w1_ref[...].astype(jnp.bfloat16)
    w2b = w2_ref[...].astype(jnp.bfloat16)
    h = jnp.dot(xb, w1b, preferred_element_type=jnp.float32)
    hb = jnp.maximum(h + b1_ref[...], 0.0).astype(jnp.bfloat16)
    y = jnp.dot(hb, w2b, preferred_element_type=jnp.float32)
    o_ref[...] = (y + b2_ref[...]).astype(o_ref.dtype)


@jax.jit
def kernel(x, w1_t, b1_r, w2_t, b2_r):
    b, n_feature = x.shape
    n_hidden, n_output = w2_t.shape
    b1_f = b1_r.astype(jnp.float32)
    b2_f = b2_r.astype(jnp.float32)

    tb = min(_TB, max(8, _cdiv(b, 8) * 8))
    nb = _cdiv(b, tb)
    b_pad = nb * tb
    if b_pad != b:
        x = jnp.pad(x, ((0, b_pad - b), (0, 0)))

    out = pl.pallas_call(
        _mlp_kernel,
        out_shape=jax.ShapeDtypeStruct((b_pad, n_output), x.dtype),
        grid=(nb,),
        in_specs=[
            pl.BlockSpec((tb, n_feature), lambda i: (i, 0)),
            pl.BlockSpec((n_feature, n_hidden), lambda i: (0, 0)),
            pl.BlockSpec((1, n_hidden), lambda i: (0, 0)),
            pl.BlockSpec((n_hidden, n_output), lambda i: (0, 0)),
            pl.BlockSpec((1, n_output), lambda i: (0, 0)),
        ],
        out_specs=pl.BlockSpec((tb, n_output), lambda i: (i, 0)),
        compiler_params=pltpu.CompilerParams(
            dimension_semantics=("arbitrary",),
            vmem_limit_bytes=int(64 * 1024 * 1024 * 0.92)),
    )(x, w1_t, b1_f, w2_t, b2_f)

    if b_pad != b:
        out = out[:b]
    return out
